# Initial kernel scaffold; baseline (speedup 1.0000x reference)
import jax
import jax.numpy as jnp
from jax.experimental import pallas as pl


def kernel(x, x_fields, x_positions, token_table, field_table, pos_table):
    raise NotImplementedError("write your pallas kernel here")


# speedup vs baseline: 3.3202x; 3.3202x over previous
import functools

import jax
import jax.numpy as jnp
from jax import lax
from jax.experimental import pallas as pl
from jax.experimental.pallas import tpu as pltpu
from jax.experimental.pallas import tpu_sc as plsc

BATCH = 4096
SEQ = 200
EMBED = 32

N_TOTAL = BATCH * SEQ
IDX_W = 128
N_IDX_ROWS = N_TOTAL // IDX_W
NW = 32
ROWS_PER_W = N_IDX_ROWS // NW
CHUNK_IR = 4
CHUNK = CHUNK_IR * IDX_W
N_CHUNKS = ROWS_PER_W // CHUNK_IR


def _sc_body(x_hbm, xf_hbm, xp_hbm, tok_hbm, fld_hbm, pos_hbm, out_hbm,
             idx_t, idx_f, idx_p, tok_v, fld_v, pos_v, sem, osem):
    wid = lax.axis_index("s") * 2 + lax.axis_index("c")
    ir_base = wid * ROWS_PER_W

    def chunk_body(g, carry):
        ir0 = ir_base + g * CHUNK_IR
        row0 = ir0 * IDX_W

        pltpu.sync_copy(x_hbm.at[pl.ds(ir0, CHUNK_IR)], idx_t)
        pltpu.sync_copy(xf_hbm.at[pl.ds(ir0, CHUNK_IR)], idx_f)
        pltpu.sync_copy(xp_hbm.at[pl.ds(ir0, CHUNK_IR)], idx_p)

        copies = []
        for j in range(CHUNK_IR):
            dst = pl.ds(j * IDX_W, IDX_W)
            copies.append(pltpu.async_copy(
                tok_hbm.at[idx_t.at[j]], tok_v.at[dst], sem))
            copies.append(pltpu.async_copy(
                fld_hbm.at[idx_f.at[j]], fld_v.at[dst], sem))
            copies.append(pltpu.async_copy(
                pos_hbm.at[idx_p.at[j]], pos_v.at[dst], sem))
        for c in copies:
            c.wait()

        def row_body(r, c2):
            for h in range(EMBED // 16):
                sl = pl.ds(h * 16, 16)
                tok_v[r, sl] = (tok_v[r, sl] + fld_v[r, sl]) + pos_v[r, sl]
            return c2

        lax.fori_loop(0, CHUNK, row_body, 0)

        pltpu.async_copy(tok_v, out_hbm.at[pl.ds(row0, CHUNK)], osem).wait()
        return carry

    lax.fori_loop(0, N_CHUNKS, chunk_body, 0)


@jax.jit
def _run(x2d, xf2d, xp2d, token_table, field_table, pos_table):
    mesh = plsc.VectorSubcoreMesh(core_axis_name="c", subcore_axis_name="s")
    f = pl.kernel(
        _sc_body,
        mesh=mesh,
        compiler_params=pltpu.CompilerParams(use_tc_tiling_on_sc=False),
        out_type=jax.ShapeDtypeStruct((N_TOTAL, EMBED), jnp.float32),
        scratch_types=[
            pltpu.VMEM((CHUNK_IR, IDX_W), jnp.int32),
            pltpu.VMEM((CHUNK_IR, IDX_W), jnp.int32),
            pltpu.VMEM((CHUNK_IR, IDX_W), jnp.int32),
            pltpu.VMEM((CHUNK, EMBED), jnp.float32),
            pltpu.VMEM((CHUNK, EMBED), jnp.float32),
            pltpu.VMEM((CHUNK, EMBED), jnp.float32),
            pltpu.SemaphoreType.DMA,
            pltpu.SemaphoreType.DMA,
        ],
    )
    return f(x2d, xf2d, xp2d, token_table, field_table, pos_table)


def kernel(x, x_fields, x_positions, token_table, field_table, pos_table):
    x2d = x.reshape(N_IDX_ROWS, IDX_W).astype(jnp.int32)
    xf2d = x_fields.reshape(N_IDX_ROWS, IDX_W).astype(jnp.int32)
    xp2d = x_positions.reshape(N_IDX_ROWS, IDX_W).astype(jnp.int32)
    out = _run(x2d, xf2d, xp2d, token_table, field_table, pos_table)
    return out.reshape(BATCH, SEQ, EMBED)


# speedup vs baseline: 5.3414x; 1.6087x over previous
import jax
import jax.numpy as jnp
from jax import lax
from jax.experimental import pallas as pl
from jax.experimental.pallas import tpu as pltpu
from jax.experimental.pallas import tpu_sc as plsc

BATCH = 4096
SEQ = 200
EMBED = 32
N_FIELDS = 69
MAXSEQLEN = 200
NF_PAD = 80
N_COMB = NF_PAD * MAXSEQLEN

N_TOTAL = BATCH * SEQ
IDX_W = 128
N_IDX_ROWS = N_TOTAL // IDX_W
NW = 32
ROWS_PER_W = N_IDX_ROWS // NW
CHUNK_IR = 4
CHUNK = CHUNK_IR * IDX_W
N_CHUNKS = ROWS_PER_W // CHUNK_IR


def _sc_body(x_hbm, xc_hbm, tok_hbm, fld_hbm, pos_hbm, out_hbm,
             fld_v, pos_v, comb_stage, comb_sp,
             idx_t0, idx_t1, idx_c0, idx_c1,
             tok_v0, tok_v1, fp_v0, fp_v1,
             gsem0, gsem1, ssem0, ssem1):
    c = lax.axis_index("c")
    s = lax.axis_index("s")
    wid = s * 2 + c
    ir_base = wid * ROWS_PER_W

    idx_t = (idx_t0, idx_t1)
    idx_c = (idx_c0, idx_c1)
    tok_v = (tok_v0, tok_v1)
    fp_v = (fp_v0, fp_v1)
    gsem = (gsem0, gsem1)
    ssem = (ssem0, ssem1)

    pltpu.sync_copy(fld_hbm, fld_v.at[pl.ds(0, N_FIELDS)])
    pltpu.sync_copy(pos_hbm, pos_v)
    for k in range(NF_PAD // 16):
        f = s + k * 16
        t0 = fld_v[f, pl.ds(0, 16)]
        t1 = fld_v[f, pl.ds(16, 16)]

        def build_body(i, carry, t0=t0, t1=t1):
            p0 = i * 4
            for dp in range(4):
                p = p0 + dp
                comb_stage[p, pl.ds(0, 16)] = t0 + pos_v[p, pl.ds(0, 16)]
                comb_stage[p, pl.ds(16, 16)] = t1 + pos_v[p, pl.ds(16, 16)]
            return carry

        lax.fori_loop(0, MAXSEQLEN // 4, build_body, 0)
        pltpu.sync_copy(comb_stage, comb_sp.at[pl.ds(f * MAXSEQLEN, MAXSEQLEN)])

    plsc.subcore_barrier()

    def fire(g, b):
        ir0 = ir_base + g * CHUNK_IR
        pltpu.sync_copy(x_hbm.at[pl.ds(ir0, CHUNK_IR)], idx_t[b])
        pltpu.sync_copy(xc_hbm.at[pl.ds(ir0, CHUNK_IR)], idx_c[b])
        for j in range(CHUNK_IR):
            dst = pl.ds(j * IDX_W, IDX_W)
            pltpu.async_copy(tok_hbm.at[idx_t[b].at[j]], tok_v[b].at[dst], gsem[b])

    def wait_gathers(b):
        for j in range(CHUNK_IR):
            dst = pl.ds(j * IDX_W, IDX_W)
            pltpu.sync_copy(comb_sp.at[idx_c[b].at[j]], fp_v[b].at[dst])
        for j in range(CHUNK_IR):
            dst = pl.ds(j * IDX_W, IDX_W)
            pltpu.make_async_copy(tok_hbm.at[idx_t[b].at[j]], tok_v[b].at[dst], gsem[b]).wait()

    def wait_scatter(b):
        pltpu.make_async_copy(tok_v[b], out_hbm.at[pl.ds(0, CHUNK)], ssem[b]).wait()

    def compute(b):
        tv, fv = tok_v[b], fp_v[b]

        def body(i, carry):
            r0 = i * 4
            for dr in range(4):
                r = r0 + dr
                for h in range(2):
                    sl = pl.ds(h * 16, 16)
                    tv[r, sl] = tv[r, sl] + fv[r, sl]
            return carry

        lax.fori_loop(0, CHUNK // 4, body, 0)

    def scatter(g, b):
        row0 = (ir_base + g * CHUNK_IR) * IDX_W
        pltpu.async_copy(tok_v[b], out_hbm.at[pl.ds(row0, CHUNK)], ssem[b])

    fire(0, 0)

    def pair_body(gp, carry):
        g0 = gp * 2
        @pl.when(gp > 0)
        def _():
            wait_scatter(1)
        fire(g0 + 1, 1)
        wait_gathers(0)
        compute(0)
        scatter(g0, 0)
        @pl.when(gp < N_CHUNKS // 2 - 1)
        def _():
            wait_scatter(0)
            fire(g0 + 2, 0)
        wait_gathers(1)
        compute(1)
        scatter(g0 + 1, 1)
        return carry

    lax.fori_loop(0, N_CHUNKS // 2, pair_body, 0)
    wait_scatter(0)
    wait_scatter(1)


@jax.jit
def _run(x2d, xc2d, token_table, field_table, pos_table):
    mesh = plsc.VectorSubcoreMesh(core_axis_name="c", subcore_axis_name="s")
    f = pl.kernel(
        _sc_body,
        mesh=mesh,
        compiler_params=pltpu.CompilerParams(use_tc_tiling_on_sc=False),
        out_type=jax.ShapeDtypeStruct((N_TOTAL, EMBED), jnp.float32),
        scratch_types=[
            pltpu.VMEM((NF_PAD, EMBED), jnp.float32),
            pltpu.VMEM((MAXSEQLEN, EMBED), jnp.float32),
            pltpu.VMEM((MAXSEQLEN, EMBED), jnp.float32),
            pltpu.VMEM_SHARED((N_COMB, EMBED), jnp.float32),
            pltpu.VMEM((CHUNK_IR, IDX_W), jnp.int32),
            pltpu.VMEM((CHUNK_IR, IDX_W), jnp.int32),
            pltpu.VMEM((CHUNK_IR, IDX_W), jnp.int32),
            pltpu.VMEM((CHUNK_IR, IDX_W), jnp.int32),
            pltpu.VMEM((CHUNK, EMBED), jnp.float32),
            pltpu.VMEM((CHUNK, EMBED), jnp.float32),
            pltpu.VMEM((CHUNK, EMBED), jnp.float32),
            pltpu.VMEM((CHUNK, EMBED), jnp.float32),
            pltpu.SemaphoreType.DMA,
            pltpu.SemaphoreType.DMA,
            pltpu.SemaphoreType.DMA,
            pltpu.SemaphoreType.DMA,
        ],
    )
    return f(x2d, xc2d, token_table, field_table, pos_table)


def kernel(x, x_fields, x_positions, token_table, field_table, pos_table):
    x2d = x.reshape(N_IDX_ROWS, IDX_W).astype(jnp.int32)
    xc2d = (x_fields.astype(jnp.int32) * MAXSEQLEN
            + x_positions.astype(jnp.int32)).reshape(N_IDX_ROWS, IDX_W)
    out = _run(x2d, xc2d, token_table, field_table, pos_table)
    return out.reshape(BATCH, SEQ, EMBED)
